# baseline (speedup 1.0000x reference)
import functools

import jax
import jax.numpy as jnp
from jax import lax
from jax.experimental import pallas as pl
from jax.experimental.pallas import tpu as pltpu
from jax.experimental.pallas import tpu_sc as plsc

HIDDEN = 16


@functools.cache
def _build(B: int, V: int, H: int):
    info = plsc.get_sparse_core_info()
    nc, ns, L = info.num_cores, info.num_subcores, info.num_lanes
    nw = nc * ns
    assert B % (L * nw) == 0
    b_per_w = B // nw
    n_chunks = b_per_w // L
    mesh = plsc.VectorSubcoreMesh(core_axis_name="c", subcore_axis_name="s")

    @functools.partial(
        pl.kernel,
        mesh=mesh,
        out_type=jax.ShapeDtypeStruct((B * H,), jnp.float32),
        scratch_types=[
            pltpu.VMEM((b_per_w,), jnp.int32),
            pltpu.VMEM((L * H,), jnp.float32),
            pltpu.SemaphoreType.DMA,
            pltpu.SemaphoreType.DMA,
        ],
    )
    def gather_kernel(table_hbm, idx_hbm, out_hbm, idx_v, stage_v, gsem, wsem):
        wid = lax.axis_index("s") * nc + lax.axis_index("c")
        base = wid * b_per_w
        pltpu.sync_copy(idx_hbm.at[pl.ds(base, b_per_w)], idx_v)

        def chunk(c, carry):
            lvec = idx_v[pl.ds(c * L, L)]
            copies = []
            for j in range(L):
                l = lvec[j]
                copies.append(
                    pltpu.async_copy(
                        table_hbm.at[pl.ds(l * H, H)],
                        stage_v.at[pl.ds(j * H, H)],
                        gsem,
                    )
                )
            for cp in copies:
                cp.wait()
            pltpu.async_copy(
                stage_v, out_hbm.at[pl.ds((base + c * L) * H, L * H)], wsem
            ).wait()
            return carry

        lax.fori_loop(0, n_chunks, chunk, 0)

    return gather_kernel


def kernel(labels, embedding_table):
    B = labels.shape[0]
    V, H = embedding_table.shape
    fn = _build(B, V, H)
    out = fn(embedding_table.reshape(V * H), labels.astype(jnp.int32))
    return out.reshape(B, H)


# speedup vs baseline: 5.4294x; 5.4294x over previous
import functools

import jax
import jax.numpy as jnp
from jax import lax
from jax.experimental import pallas as pl
from jax.experimental.pallas import tpu as pltpu
from jax.experimental.pallas import tpu_sc as plsc

HIDDEN = 16


@functools.cache
def _build(B: int, V: int, H: int):
    info = plsc.get_sparse_core_info()
    nc, ns, L = info.num_cores, info.num_subcores, info.num_lanes
    nw = nc * ns
    assert B % (L * nw) == 0
    b_per_w = B // nw
    n_groups = b_per_w // 128
    mesh = plsc.VectorSubcoreMesh(core_axis_name="c", subcore_axis_name="s")

    @functools.partial(
        pl.kernel,
        mesh=mesh,
        compiler_params=pltpu.CompilerParams(needs_layout_passes=False),
        out_type=jax.ShapeDtypeStruct((H, B), jnp.float32),
        scratch_types=[
            pltpu.VMEM((b_per_w,), jnp.int32),
            pltpu.VMEM((L, H, 128), jnp.float32),
            pltpu.VMEM((H, 128), jnp.float32),
            pltpu.SemaphoreType.DMA,
            pltpu.SemaphoreType.DMA,
        ],
    )
    def gather_kernel(table_hbm, idx_hbm, out_hbm, idx_v, blk_v, outg_v, gsem, wsem):
        wid = lax.axis_index("s") * nc + lax.axis_index("c")
        base = wid * b_per_w
        pltpu.sync_copy(idx_hbm.at[pl.ds(base, b_per_w)], idx_v)
        rows16 = lax.iota(jnp.int32, L)

        for g in range(n_groups):
            def chunk(c8, carry):
                lvec = idx_v[pl.ds(g * 128 + c8 * L, L)]
                copies = []
                for j in range(L):
                    l = lvec[j]
                    cb = pl.multiple_of((l >> 7) << 7, 128)
                    copies.append(
                        pltpu.async_copy(
                            table_hbm.at[:, pl.ds(cb, 128)],
                            blk_v.at[j],
                            gsem,
                        )
                    )
                for j in range(L):
                    copies[j].wait()
                    l = lvec[j]
                    col = jnp.broadcast_to(l & 127, (L,))
                    vals = plsc.load_gather(blk_v.at[j], [rows16, col])
                    dstcol = jnp.broadcast_to(c8 * L + j, (L,))
                    plsc.store_scatter(outg_v, [rows16, dstcol], vals)
                return carry

            lax.fori_loop(0, 128 // L, chunk, 0)
            pltpu.async_copy(
                outg_v, out_hbm.at[:, pl.ds(base + g * 128, 128)], wsem
            ).wait()

    return gather_kernel


def kernel(labels, embedding_table):
    B = labels.shape[0]
    V, H = embedding_table.shape
    fn = _build(B, V, H)
    out_t = fn(embedding_table.T, labels.astype(jnp.int32))
    return out_t.T


# speedup vs baseline: 7.3378x; 1.3515x over previous
import functools

import jax
import jax.numpy as jnp
from jax import lax
from jax.experimental import pallas as pl
from jax.experimental.pallas import tpu as pltpu
from jax.experimental.pallas import tpu_sc as plsc

HIDDEN = 16


@functools.cache
def _build(B: int, V: int, H: int):
    info = plsc.get_sparse_core_info()
    nc, ns, L = info.num_cores, info.num_subcores, info.num_lanes
    nw = nc * ns
    assert B % (L * nw) == 0
    b_per_w = B // nw
    n_chunks = b_per_w // L
    n_pairs = n_chunks // 2
    mesh = plsc.VectorSubcoreMesh(core_axis_name="c", subcore_axis_name="s")

    @functools.partial(
        pl.kernel,
        mesh=mesh,
        compiler_params=pltpu.CompilerParams(needs_layout_passes=False),
        out_type=jax.ShapeDtypeStruct((H, B), jnp.float32),
        scratch_types=[
            pltpu.VMEM((b_per_w,), jnp.int32),
            pltpu.VMEM((2, L, H, 128), jnp.float32),
            pltpu.VMEM((H, 128), jnp.float32),
            pltpu.SemaphoreType.DMA,
            pltpu.SemaphoreType.DMA,
        ],
    )
    def gather_kernel(table_hbm, idx_hbm, out_hbm, idx_v, blk_v, outg_v, gsem, wsem):
        wid = lax.axis_index("s") * nc + lax.axis_index("c")
        base = wid * b_per_w
        pltpu.sync_copy(idx_hbm.at[pl.ds(base, b_per_w)], idx_v)
        rows16 = lax.iota(jnp.int32, L)

        def issue(c, buf):
            lvec = idx_v[pl.ds(c * L, L)]
            for j in range(L):
                cb = pl.multiple_of((lvec[j] >> 7) << 7, 128)
                pltpu.async_copy(
                    table_hbm.at[:, pl.ds(cb, 128)], blk_v.at[buf].at[j], gsem
                )

        def drain_and_extract(c, buf):
            lvec = idx_v[pl.ds(c * L, L)]
            for j in range(L):
                pltpu.make_async_copy(
                    table_hbm.at[:, pl.ds(0, 128)], blk_v.at[buf].at[j], gsem
                ).wait()
            for j in range(L):
                col = jnp.broadcast_to(lvec[j] & 127, (L,))
                vals = plsc.load_gather(blk_v.at[buf].at[j], [rows16, col])
                dstcol = jnp.broadcast_to((c & 7) * L + j, (L,))
                plsc.store_scatter(outg_v, [rows16, dstcol], vals)

        issue(0, 0)

        def pair_body(i, carry):
            c0 = i * 2
            c1 = c0 + 1
            issue(c1, 1)
            drain_and_extract(c0, 0)

            @pl.when(i < n_pairs - 1)
            def _():
                issue(c0 + 2, 0)

            drain_and_extract(c1, 1)

            @pl.when((i & 3) == 3)
            def _():
                g = pl.multiple_of(base + ((c1 >> 3) << 7), 128)
                pltpu.async_copy(
                    outg_v, out_hbm.at[:, pl.ds(g, 128)], wsem
                ).wait()

            return carry

        lax.fori_loop(0, n_pairs, pair_body, 0)

    return gather_kernel


def kernel(labels, embedding_table):
    B = labels.shape[0]
    V, H = embedding_table.shape
    fn = _build(B, V, H)
    out_t = fn(embedding_table.T, labels.astype(jnp.int32))
    return out_t.T


# speedup vs baseline: 7.3467x; 1.0012x over previous
import functools

import jax
import jax.numpy as jnp
from jax import lax
from jax.experimental import pallas as pl
from jax.experimental.pallas import tpu as pltpu
from jax.experimental.pallas import tpu_sc as plsc

HIDDEN = 16


@functools.cache
def _build(B: int, V: int, H: int):
    info = plsc.get_sparse_core_info()
    nc, ns, L = info.num_cores, info.num_subcores, info.num_lanes
    nw = nc * ns
    assert B % (L * nw) == 0
    b_per_w = B // nw
    n_chunks = b_per_w // L
    n_pairs = n_chunks // 2
    mesh = plsc.VectorSubcoreMesh(core_axis_name="c", subcore_axis_name="s")

    @functools.partial(
        pl.kernel,
        mesh=mesh,
        compiler_params=pltpu.CompilerParams(needs_layout_passes=False),
        out_type=jax.ShapeDtypeStruct((H, B), jnp.float32),
        scratch_types=[
            pltpu.VMEM((b_per_w,), jnp.int32),
            pltpu.VMEM((2, L, H, 128), jnp.float32),
            pltpu.VMEM((2, H, 128), jnp.float32),
            pltpu.SemaphoreType.DMA,
            pltpu.SemaphoreType.DMA,
        ],
    )
    def gather_kernel(table_hbm, idx_hbm, out_hbm, idx_v, blk_v, outg_v, gsem, wsem):
        wid = lax.axis_index("s") * nc + lax.axis_index("c")
        base = wid * b_per_w
        pltpu.sync_copy(idx_hbm.at[pl.ds(base, b_per_w)], idx_v)
        rows16 = lax.iota(jnp.int32, L)

        def issue(c, buf):
            lvec = idx_v[pl.ds(c * L, L)]
            for j in range(L):
                cb = pl.multiple_of((lvec[j] >> 7) << 7, 128)
                pltpu.async_copy(
                    table_hbm.at[:, pl.ds(cb, 128)], blk_v.at[buf].at[j], gsem
                )

        def drain_and_extract(c, buf):
            lvec = idx_v[pl.ds(c * L, L)]
            pltpu.make_async_copy(
                table_hbm.at[:, pl.ds(0, 128)], blk_v.at[buf], gsem
            ).wait()
            colv = lvec & 127
            cb8 = (c & 7) * L
            for j in range(L):
                col = jnp.broadcast_to(colv[j], (L,))
                vals = plsc.load_gather(blk_v.at[buf].at[j], [rows16, col])
                dstcol = jnp.broadcast_to(cb8 + j, (L,))
                plsc.store_scatter(outg_v.at[(c >> 3) & 1], [rows16, dstcol], vals)

        issue(0, 0)

        def pair_body(i, carry):
            c0 = i * 2
            c1 = c0 + 1
            issue(c1, 1)
            drain_and_extract(c0, 0)

            @pl.when(i < n_pairs - 1)
            def _():
                issue(c0 + 2, 0)

            drain_and_extract(c1, 1)

            @pl.when((i & 3) == 3)
            def _():
                gi = c1 >> 3
                g = pl.multiple_of(base + (gi << 7), 128)

                @pl.when(i > 3)
                def _():
                    pltpu.make_async_copy(
                        outg_v.at[(gi + 1) & 1],
                        out_hbm.at[:, pl.ds(base, 128)],
                        wsem,
                    ).wait()

                pltpu.async_copy(
                    outg_v.at[gi & 1], out_hbm.at[:, pl.ds(g, 128)], wsem
                )

            return carry

        lax.fori_loop(0, n_pairs, pair_body, 0)
        pltpu.make_async_copy(
            outg_v.at[0], out_hbm.at[:, pl.ds(base, 128)], wsem
        ).wait()

    return gather_kernel


def kernel(labels, embedding_table):
    B = labels.shape[0]
    V, H = embedding_table.shape
    fn = _build(B, V, H)
    out_t = fn(embedding_table.T, labels.astype(jnp.int32))
    return out_t.T


# speedup vs baseline: 7.3728x; 1.0035x over previous
import functools

import jax
import jax.numpy as jnp
from jax import lax
from jax.experimental import pallas as pl
from jax.experimental.pallas import tpu as pltpu
from jax.experimental.pallas import tpu_sc as plsc

HIDDEN = 16


@functools.cache
def _build(B: int, V: int, H: int):
    info = plsc.get_sparse_core_info()
    nc, ns, L = info.num_cores, info.num_subcores, info.num_lanes
    nw = nc * ns
    assert B % (L * nw) == 0
    b_per_w = B // nw
    n_chunks = b_per_w // L
    mesh = plsc.VectorSubcoreMesh(core_axis_name="c", subcore_axis_name="s")

    @functools.partial(
        pl.kernel,
        mesh=mesh,
        compiler_params=pltpu.CompilerParams(needs_layout_passes=False),
        out_type=jax.ShapeDtypeStruct((H, B), jnp.float32),
        scratch_types=[
            pltpu.VMEM((b_per_w,), jnp.int32),
            pltpu.VMEM((3, L, H, 128), jnp.float32),
            pltpu.VMEM((2, H, 128), jnp.float32),
            pltpu.SemaphoreType.DMA,
            pltpu.SemaphoreType.DMA,
        ],
    )
    def gather_kernel(table_hbm, idx_hbm, out_hbm, idx_v, blk_v, outg_v, gsem, wsem):
        wid = lax.axis_index("s") * nc + lax.axis_index("c")
        base = wid * b_per_w
        pltpu.sync_copy(idx_hbm.at[pl.ds(base, b_per_w)], idx_v)
        rows16 = lax.iota(jnp.int32, L)

        def issue(c, buf):
            lvec = idx_v[pl.ds(c * L, L)]
            for j in range(L):
                cb = pl.multiple_of((lvec[j] >> 7) << 7, 128)
                pltpu.async_copy(
                    table_hbm.at[:, pl.ds(cb, 128)], blk_v.at[buf].at[j], gsem
                )

        def drain_and_extract(c, buf):
            lvec = idx_v[pl.ds(c * L, L)]
            pltpu.make_async_copy(
                table_hbm.at[:, pl.ds(0, 128)], blk_v.at[buf], gsem
            ).wait()
            colv = lvec & 127
            cb8 = (c & 7) * L
            for j in range(L):
                col = jnp.broadcast_to(colv[j], (L,))
                vals = plsc.load_gather(blk_v.at[buf].at[j], [rows16, col])
                dstcol = jnp.broadcast_to(cb8 + j, (L,))
                plsc.store_scatter(outg_v.at[(c >> 3) & 1], [rows16, dstcol], vals)

        issue(0, 0)
        issue(1, 1)

        def body(c, carry):
            buf = carry

            @pl.when(c < n_chunks - 2)
            def _():
                issue(c + 2, buf)

            cur = buf + 1 - jnp.where(buf >= 2, 3, 0)
            drain_and_extract(c, cur)

            @pl.when((c & 7) == 7)
            def _():
                gi = c >> 3
                g = pl.multiple_of(base + (gi << 7), 128)

                @pl.when(c > 8)
                def _():
                    pltpu.make_async_copy(
                        outg_v.at[(gi + 1) & 1],
                        out_hbm.at[:, pl.ds(base, 128)],
                        wsem,
                    ).wait()

                pltpu.async_copy(
                    outg_v.at[gi & 1], out_hbm.at[:, pl.ds(g, 128)], wsem
                )

            return cur

        lax.fori_loop(0, n_chunks, body, jnp.int32(2))
        pltpu.make_async_copy(
            outg_v.at[0], out_hbm.at[:, pl.ds(base, 128)], wsem
        ).wait()

    return gather_kernel


def kernel(labels, embedding_table):
    B = labels.shape[0]
    V, H = embedding_table.shape
    fn = _build(B, V, H)
    out_t = fn(embedding_table.T, labels.astype(jnp.int32))
    return out_t.T
